# baseline (speedup 1.0000x reference)
import functools

import jax
import jax.numpy as jnp
from jax import lax
from jax.experimental import pallas as pl
from jax.experimental.pallas import tpu as pltpu
from jax.experimental.pallas import tpu_sc as plsc

NE = 8
DM = 1024
DB = 64
DBP = 128
DIN = DM + DB
DFF = 2048
T = 8192
BLK = 256
CAP = T + NE * BLK
NBLK = CAP // BLK
NBLK_PAD = 48

L = 16
NC = 2
NS = 16
NW = NC * NS
TPW = T // NW
CH = 64
NCH = TPW // CH
CH2 = 32
NCH2 = TPW // CH2

_sc_mesh = plsc.VectorSubcoreMesh(core_axis_name="c", subcore_axis_name="s")


@functools.partial(
    pl.kernel,
    out_type=(
        jax.ShapeDtypeStruct((CAP, DM), jnp.float32),
        jax.ShapeDtypeStruct((CAP, DBP), jnp.float32),
        jax.ShapeDtypeStruct((T,), jnp.int32),
        jax.ShapeDtypeStruct((NBLK_PAD,), jnp.int32),
    ),
    mesh=_sc_mesh,
    compiler_params=pltpu.CompilerParams(needs_layout_passes=False),
    scratch_types=[
        pltpu.VMEM((2 * TPW,), jnp.int32),
        pltpu.VMEM((NCH2, CH2), jnp.int32),
        pltpu.VMEM((NCH2, CH2), jnp.int32),
        pltpu.VMEM((L,), jnp.int32),
        pltpu.VMEM((NW * L,), jnp.int32),
        pltpu.VMEM((L,), jnp.int32),
        pltpu.VMEM((L,), jnp.int32),
        pltpu.VMEM((NBLK_PAD,), jnp.int32),
        pltpu.VMEM((CH2, DBP), jnp.float32),
        pltpu.VMEM((CH2, DM), jnp.float32),
        pltpu.VMEM((CH2, DM), jnp.float32),
        pltpu.VMEM_SHARED((NW * L,), jnp.int32),
        pltpu.SemaphoreType.DMA,
        pltpu.SemaphoreType.DMA,
        pltpu.SemaphoreType.DMA,
        pltpu.SemaphoreType.DMA,
        pltpu.SemaphoreType.DMA,
    ],
)
def _sc_dispatch(hid_hbm, pos_hbm, beh_hbm, bt_hbm,
                 xh_hbm, xb_hbm, dest_hbm, bexp_hbm,
                 pos_v, behi_v, dest_v, cnt_v, allcnt_v, base_v, ends_v,
                 bexp_v, behrows_v, hidrows0, hidrows1, counts_sh,
                 sem, rsem0, rsem1, wsem0, wsem1):
    c = lax.axis_index("c")
    s = lax.axis_index("s")
    g = s * 2 + c
    lanes = lax.broadcasted_iota(jnp.int32, (L,), 0)
    zeros = jnp.zeros((L,), jnp.int32)

    pltpu.sync_copy(pos_hbm.at[pl.ds(s * 2 * TPW, 2 * TPW)], pos_v)
    for half in range(2):
        cnt = zeros
        for i in range(TPW // L):
            p = pos_v[pl.ds(half * TPW + i * L, L)]
            for e in range(NE):
                cpop = jnp.sum((p == e).astype(jnp.int32))
                cnt = cnt + jnp.where(
                    lanes == e, lax.broadcast_in_dim(cpop, (L,), ()), 0)
        cnt_v[...] = cnt
        pltpu.sync_copy(cnt_v, counts_sh.at[pl.ds((2 * s + half) * L, L)])
    plsc.subcore_barrier()
    pltpu.sync_copy(counts_sh, allcnt_v)

    gv = lax.broadcast_in_dim(g, (L,), ())
    total = zeros
    prior = zeros
    for r in range(NW):
        row = allcnt_v[pl.ds(r * L, L)]
        total = total + row
        prior = prior + jnp.where(jnp.full((L,), r, jnp.int32) < gv, row, zeros)

    pcnt = ((total + (BLK - 1)) >> 8) << 8
    ends = plsc.cumsum(pcnt)
    starts = ends - pcnt
    base_v[...] = starts + prior
    ends_v[...] = ends

    for i in range(TPW // L):
        p = pos_v[pl.ds(c * TPW + i * L, L)]
        bofp = plsc.load_gather(base_v, [p])
        rank = zeros
        for e in range(NE):
            m = p == e
            cs = plsc.cumsum(m.astype(jnp.int32))
            rank = jnp.where(m, cs - 1, rank)
            cpop = lax.broadcast_in_dim(jnp.sum(m.astype(jnp.int32)), (L,), ())
            plsc.addupdate_scatter(
                base_v, [jnp.full((L,), e, jnp.int32)], cpop, mask=lanes == 0)
        dest_v[i // (CH2 // L), pl.ds((i % (CH2 // L)) * L, L)] = bofp + rank

    tok0 = g * TPW
    for j in range(NCH2):
        pltpu.sync_copy(dest_v.at[j], dest_hbm.at[pl.ds(tok0 + j * CH2, CH2)])
        pltpu.sync_copy(beh_hbm.at[pl.ds(tok0 + j * CH2, CH2)], behi_v.at[j])
        pltpu.async_copy(bt_hbm.at[behi_v.at[j]], behrows_v, sem).wait()
        pltpu.async_copy(behrows_v, xb_hbm.at[dest_v.at[j]], sem).wait()

    bufs = (hidrows0, hidrows1)
    rsems = (rsem0, rsem1)
    wsems = (wsem0, wsem1)
    reads = [None] * NCH2
    writes = [None] * NCH2
    reads[0] = pltpu.async_copy(
        hid_hbm.at[pl.ds(tok0, CH2)], bufs[0], rsems[0])
    for j in range(NCH2):
        b = j % 2
        reads[j].wait()
        if j + 1 < NCH2:
            if j >= 1:
                writes[j - 1].wait()
            reads[j + 1] = pltpu.async_copy(
                hid_hbm.at[pl.ds(tok0 + (j + 1) * CH2, CH2)],
                bufs[(j + 1) % 2], rsems[(j + 1) % 2])
        writes[j] = pltpu.async_copy(
            bufs[b], xh_hbm.at[dest_v.at[j]], wsems[b])
    writes[NCH2 - 2].wait()
    writes[NCH2 - 1].wait()

    for k in range(NBLK_PAD // L):
        blk = (lanes + k * L) * BLK
        cntk = zeros
        for e in range(NE):
            ende = jnp.take_along_axis(
                ends, jnp.full((L,), e, jnp.int32), axis=0,
                mode="promise_in_bounds")
            cntk = cntk + jnp.where(ende <= blk, 1, 0)
        bexp_v[pl.ds(k * L, L)] = jnp.minimum(cntk, NE - 1)

    @pl.when((c == 0) & (s == 0))
    def _():
        pltpu.sync_copy(bexp_v, bexp_hbm)


@functools.partial(
    pl.kernel,
    out_type=jax.ShapeDtypeStruct((T, DM), jnp.float32),
    mesh=_sc_mesh,
    compiler_params=pltpu.CompilerParams(needs_layout_passes=False),
    scratch_types=[
        pltpu.VMEM((NCH2, CH2), jnp.int32),
        pltpu.VMEM((CH2, DM), jnp.float32),
        pltpu.VMEM((CH2, DM), jnp.float32),
        pltpu.SemaphoreType.DMA,
        pltpu.SemaphoreType.DMA,
        pltpu.SemaphoreType.DMA,
        pltpu.SemaphoreType.DMA,
    ],
)
def _sc_combine(osort_hbm, dest_hbm, out_hbm, dest_v, rows0, rows1,
                rsem0, rsem1, wsem0, wsem1):
    c = lax.axis_index("c")
    s = lax.axis_index("s")
    tok0 = (s * 2 + c) * TPW
    for j in range(NCH2):
        pltpu.sync_copy(
            dest_hbm.at[pl.ds(tok0 + j * CH2, CH2)], dest_v.at[j])
    bufs = (rows0, rows1)
    rsems = (rsem0, rsem1)
    wsems = (wsem0, wsem1)
    reads = [None] * NCH2
    writes = [None] * NCH2
    reads[0] = pltpu.async_copy(
        osort_hbm.at[dest_v.at[0]], bufs[0], rsems[0])
    for j in range(NCH2):
        b = j % 2
        reads[j].wait()
        if j + 1 < NCH2:
            if j >= 1:
                writes[j - 1].wait()
            reads[j + 1] = pltpu.async_copy(
                osort_hbm.at[dest_v.at[j + 1]],
                bufs[(j + 1) % 2], rsems[(j + 1) % 2])
        writes[j] = pltpu.async_copy(
            bufs[b], out_hbm.at[pl.ds(tok0 + j * CH2, CH2)], wsems[b])
    writes[NCH2 - 2].wait()
    writes[NCH2 - 1].wait()


def _mlp_body(be_ref, xh_ref, xb_ref, wg_ref, wu_ref, wd_ref, o_ref):
    del be_ref
    xh = xh_ref[...]
    xb = xb_ref[:, :DB]
    wg = wg_ref[0]
    wu = wu_ref[0]
    wd = wd_ref[0]
    g = jnp.dot(xh, wg[:DM], preferred_element_type=jnp.float32)
    g = g + jnp.dot(xb, wg[DM:], preferred_element_type=jnp.float32)
    u = jnp.dot(xh, wu[:DM], preferred_element_type=jnp.float32)
    u = u + jnp.dot(xb, wu[DM:], preferred_element_type=jnp.float32)
    a = g * jax.nn.sigmoid(g) * u
    o_ref[...] = jnp.dot(a, wd, preferred_element_type=jnp.float32)


def _grouped_mlp(block_expert, x_hid, x_beh, wg, wu, wd):
    grid_spec = pltpu.PrefetchScalarGridSpec(
        num_scalar_prefetch=1,
        grid=(NBLK,),
        in_specs=[
            pl.BlockSpec((BLK, DM), lambda b, be: (b, 0)),
            pl.BlockSpec((BLK, DBP), lambda b, be: (b, 0)),
            pl.BlockSpec((1, DIN, DFF), lambda b, be: (be[b], 0, 0)),
            pl.BlockSpec((1, DIN, DFF), lambda b, be: (be[b], 0, 0)),
            pl.BlockSpec((1, DFF, DM), lambda b, be: (be[b], 0, 0)),
        ],
        out_specs=pl.BlockSpec((BLK, DM), lambda b, be: (b, 0)),
    )
    return pl.pallas_call(
        _mlp_body,
        grid_spec=grid_spec,
        out_shape=jax.ShapeDtypeStruct((CAP, DM), jnp.float32),
        compiler_params=pltpu.CompilerParams(
            vmem_limit_bytes=110 * 1024 * 1024),
    )(block_expert, x_hid, x_beh, wg, wu, wd)


def kernel(hidden_states, position_index, behavior_index, behavior_table,
           W_gate, W_up, W_down):
    pos = position_index.astype(jnp.int32)
    beh = behavior_index.astype(jnp.int32)
    bt_pad = jnp.pad(behavior_table, ((0, 0), (0, DBP - DB)))
    x_hid, x_beh, dest, block_expert = _sc_dispatch(
        hidden_states, pos, beh, bt_pad)
    out_sorted = _grouped_mlp(
        block_expert, x_hid, x_beh, W_gate, W_up, W_down)
    return _sc_combine(out_sorted, dest)


# speedup vs baseline: 1.0395x; 1.0395x over previous
import functools

import jax
import jax.numpy as jnp
from jax import lax
from jax.experimental import pallas as pl
from jax.experimental.pallas import tpu as pltpu
from jax.experimental.pallas import tpu_sc as plsc

NE = 8
DM = 1024
DB = 64
DBP = 128
DIN = DM + DB
DFF = 2048
T = 8192
BLK = 256
CAP = T + NE * BLK
NBLK = CAP // BLK
NBLK_PAD = 48

L = 16
NC = 2
NS = 16
NW = NC * NS
TPW = T // NW
CH = 64
NCH = TPW // CH
CH2 = 32
NCH2 = TPW // CH2

_sc_mesh = plsc.VectorSubcoreMesh(core_axis_name="c", subcore_axis_name="s")


@functools.partial(
    pl.kernel,
    out_type=(
        jax.ShapeDtypeStruct((CAP, DM), jnp.float32),
        jax.ShapeDtypeStruct((CAP, DBP), jnp.float32),
        jax.ShapeDtypeStruct((T,), jnp.int32),
        jax.ShapeDtypeStruct((NBLK_PAD,), jnp.int32),
        jax.ShapeDtypeStruct((L,), jnp.int32),
    ),
    mesh=_sc_mesh,
    compiler_params=pltpu.CompilerParams(needs_layout_passes=False),
    scratch_types=[
        pltpu.VMEM((2 * TPW,), jnp.int32),
        pltpu.VMEM((TPW,), jnp.int32),
        pltpu.VMEM((NCH2, CH2), jnp.int32),
        pltpu.VMEM((L,), jnp.int32),
        pltpu.VMEM((NW * L,), jnp.int32),
        pltpu.VMEM((L,), jnp.int32),
        pltpu.VMEM((L,), jnp.int32),
        pltpu.VMEM((NBLK_PAD,), jnp.int32),
        pltpu.VMEM((CH2, DBP), jnp.float32),
        pltpu.VMEM((CH2, DM), jnp.float32),
        pltpu.VMEM((CH2, DM), jnp.float32),
        pltpu.VMEM_SHARED((NW * L,), jnp.int32),
        pltpu.SemaphoreType.DMA,
        pltpu.SemaphoreType.DMA,
        pltpu.SemaphoreType.DMA,
        pltpu.SemaphoreType.DMA,
        pltpu.SemaphoreType.DMA,
    ],
)
def _sc_dispatch(hid_hbm, pos_hbm, beh_hbm, bt_hbm,
                 xh_hbm, xb_hbm, dest_hbm, bexp_hbm, meta_hbm,
                 pos_v, behi_v, dest_v, cnt_v, allcnt_v, base_v, ends_v,
                 bexp_v, behrows_v, hidrows0, hidrows1, counts_sh,
                 sem, rsem0, rsem1, wsem0, wsem1):
    c = lax.axis_index("c")
    s = lax.axis_index("s")
    g = s * 2 + c
    lanes = lax.broadcasted_iota(jnp.int32, (L,), 0)
    zeros = jnp.zeros((L,), jnp.int32)

    pltpu.sync_copy(pos_hbm.at[pl.ds(s * 2 * TPW, 2 * TPW)], pos_v)
    for half in range(2):
        cnt = zeros
        for i in range(TPW // L):
            p = pos_v[pl.ds(half * TPW + i * L, L)]
            for e in range(NE):
                cpop = jnp.sum((p == e).astype(jnp.int32))
                cnt = cnt + jnp.where(
                    lanes == e, lax.broadcast_in_dim(cpop, (L,), ()), 0)
        cnt_v[...] = cnt
        pltpu.sync_copy(cnt_v, counts_sh.at[pl.ds((2 * s + half) * L, L)])
    plsc.subcore_barrier()
    pltpu.sync_copy(counts_sh, allcnt_v)

    gv = lax.broadcast_in_dim(g, (L,), ())
    total = zeros
    prior = zeros
    for r in range(NW):
        row = allcnt_v[pl.ds(r * L, L)]
        total = total + row
        prior = prior + jnp.where(jnp.full((L,), r, jnp.int32) < gv, row, zeros)

    pcnt = ((total + (BLK - 1)) >> 8) << 8
    ends = plsc.cumsum(pcnt)
    starts = ends - pcnt
    base_v[...] = starts + prior
    ends_v[...] = ends

    for i in range(TPW // L):
        p = pos_v[pl.ds(c * TPW + i * L, L)]
        bofp = plsc.load_gather(base_v, [p])
        rank = zeros
        for e in range(NE):
            m = p == e
            cs = plsc.cumsum(m.astype(jnp.int32))
            rank = jnp.where(m, cs - 1, rank)
            cpop = lax.broadcast_in_dim(jnp.sum(m.astype(jnp.int32)), (L,), ())
            plsc.addupdate_scatter(
                base_v, [jnp.full((L,), e, jnp.int32)], cpop, mask=lanes == 0)
        dest_v[i // (CH2 // L), pl.ds((i % (CH2 // L)) * L, L)] = bofp + rank

    tok0 = g * TPW
    pltpu.sync_copy(beh_hbm.at[pl.ds(tok0, TPW)], behi_v)
    for j in range(NCH2):
        pltpu.sync_copy(dest_v.at[j], dest_hbm.at[pl.ds(tok0 + j * CH2, CH2)])
        pltpu.async_copy(
            bt_hbm.at[behi_v.at[pl.ds(j * CH2, CH2)]], behrows_v, sem).wait()
        pltpu.async_copy(behrows_v, xb_hbm.at[dest_v.at[j]], sem).wait()

    bufs = (hidrows0, hidrows1)
    rsems = (rsem0, rsem1)
    wsems = (wsem0, wsem1)
    reads = [None] * NCH2
    writes = [None] * NCH2
    reads[0] = pltpu.async_copy(
        hid_hbm.at[pl.ds(tok0, CH2)], bufs[0], rsems[0])
    for j in range(NCH2):
        b = j % 2
        reads[j].wait()
        if j + 1 < NCH2:
            if j >= 1:
                writes[j - 1].wait()
            reads[j + 1] = pltpu.async_copy(
                hid_hbm.at[pl.ds(tok0 + (j + 1) * CH2, CH2)],
                bufs[(j + 1) % 2], rsems[(j + 1) % 2])
        writes[j] = pltpu.async_copy(
            bufs[b], xh_hbm.at[dest_v.at[j]], wsems[b])
    writes[NCH2 - 2].wait()
    writes[NCH2 - 1].wait()

    for k in range(NBLK_PAD // L):
        blk = (lanes + k * L) * BLK
        cntk = zeros
        for e in range(NE):
            ende = jnp.take_along_axis(
                ends, jnp.full((L,), e, jnp.int32), axis=0,
                mode="promise_in_bounds")
            cntk = cntk + jnp.where(ende <= blk, 1, 0)
        bexp_v[pl.ds(k * L, L)] = jnp.minimum(cntk, NE - 1)

    @pl.when((c == 0) & (s == 0))
    def _():
        pltpu.sync_copy(bexp_v, bexp_hbm)
        pltpu.sync_copy(ends_v, meta_hbm)


@functools.partial(
    pl.kernel,
    out_type=jax.ShapeDtypeStruct((T, DM), jnp.float32),
    mesh=_sc_mesh,
    compiler_params=pltpu.CompilerParams(needs_layout_passes=False),
    scratch_types=[
        pltpu.VMEM((TPW,), jnp.int32),
        pltpu.VMEM((CH2, DM), jnp.float32),
        pltpu.VMEM((CH2, DM), jnp.float32),
        pltpu.SemaphoreType.DMA,
        pltpu.SemaphoreType.DMA,
        pltpu.SemaphoreType.DMA,
        pltpu.SemaphoreType.DMA,
    ],
)
def _sc_combine(osort_hbm, dest_hbm, out_hbm, dest_v, rows0, rows1,
                rsem0, rsem1, wsem0, wsem1):
    c = lax.axis_index("c")
    s = lax.axis_index("s")
    tok0 = (s * 2 + c) * TPW
    pltpu.sync_copy(dest_hbm.at[pl.ds(tok0, TPW)], dest_v)
    bufs = (rows0, rows1)
    rsems = (rsem0, rsem1)
    wsems = (wsem0, wsem1)
    reads = [None] * NCH2
    writes = [None] * NCH2
    reads[0] = pltpu.async_copy(
        osort_hbm.at[dest_v.at[pl.ds(0, CH2)]], bufs[0], rsems[0])
    for j in range(NCH2):
        b = j % 2
        reads[j].wait()
        if j + 1 < NCH2:
            if j >= 1:
                writes[j - 1].wait()
            reads[j + 1] = pltpu.async_copy(
                osort_hbm.at[dest_v.at[pl.ds((j + 1) * CH2, CH2)]],
                bufs[(j + 1) % 2], rsems[(j + 1) % 2])
        writes[j] = pltpu.async_copy(
            bufs[b], out_hbm.at[pl.ds(tok0 + j * CH2, CH2)], wsems[b])
    writes[NCH2 - 2].wait()
    writes[NCH2 - 1].wait()


def _mlp_body(be_ref, meta_ref, xh_ref, xb_ref, wg_ref, wu_ref, wd_ref,
              o_ref):
    del be_ref
    nused = meta_ref[NE - 1] >> 8

    @pl.when(pl.program_id(0) < nused)
    def _():
        _mlp_compute(xh_ref, xb_ref, wg_ref, wu_ref, wd_ref, o_ref)


def _mlp_compute(xh_ref, xb_ref, wg_ref, wu_ref, wd_ref, o_ref):
    xh = xh_ref[...].astype(jnp.bfloat16)
    xb = xb_ref[:, :DB].astype(jnp.bfloat16)
    wg = wg_ref[0].astype(jnp.bfloat16)
    wu = wu_ref[0].astype(jnp.bfloat16)
    wd = wd_ref[0].astype(jnp.bfloat16)
    g = jnp.dot(xh, wg[:DM], preferred_element_type=jnp.float32)
    g = g + jnp.dot(xb, wg[DM:], preferred_element_type=jnp.float32)
    u = jnp.dot(xh, wu[:DM], preferred_element_type=jnp.float32)
    u = u + jnp.dot(xb, wu[DM:], preferred_element_type=jnp.float32)
    a = (g * jax.nn.sigmoid(g) * u).astype(jnp.bfloat16)
    o_ref[...] = jnp.dot(a, wd, preferred_element_type=jnp.float32)


def _grouped_mlp(block_expert, meta, x_hid, x_beh, wg, wu, wd):
    grid_spec = pltpu.PrefetchScalarGridSpec(
        num_scalar_prefetch=2,
        grid=(NBLK,),
        in_specs=[
            pl.BlockSpec((BLK, DM), lambda b, be, me: (b, 0)),
            pl.BlockSpec((BLK, DBP), lambda b, be, me: (b, 0)),
            pl.BlockSpec((1, DIN, DFF), lambda b, be, me: (be[b], 0, 0)),
            pl.BlockSpec((1, DIN, DFF), lambda b, be, me: (be[b], 0, 0)),
            pl.BlockSpec((1, DFF, DM), lambda b, be, me: (be[b], 0, 0)),
        ],
        out_specs=pl.BlockSpec((BLK, DM), lambda b, be, me: (b, 0)),
    )
    return pl.pallas_call(
        _mlp_body,
        grid_spec=grid_spec,
        out_shape=jax.ShapeDtypeStruct((CAP, DM), jnp.float32),
        compiler_params=pltpu.CompilerParams(
            vmem_limit_bytes=110 * 1024 * 1024),
    )(block_expert, meta, x_hid, x_beh, wg, wu, wd)


def kernel(hidden_states, position_index, behavior_index, behavior_table,
           W_gate, W_up, W_down):
    pos = position_index.astype(jnp.int32)
    beh = behavior_index.astype(jnp.int32)
    bt_pad = jnp.pad(behavior_table, ((0, 0), (0, DBP - DB)))
    x_hid, x_beh, dest, block_expert, meta = _sc_dispatch(
        hidden_states, pos, beh, bt_pad)
    out_sorted = _grouped_mlp(
        block_expert, meta, x_hid, x_beh, W_gate, W_up, W_down)
    return _sc_combine(out_sorted, dest)


# speedup vs baseline: 1.0598x; 1.0195x over previous
import functools

import jax
import jax.numpy as jnp
from jax import lax
from jax.experimental import pallas as pl
from jax.experimental.pallas import tpu as pltpu
from jax.experimental.pallas import tpu_sc as plsc

NE = 8
DM = 1024
DB = 64
DBP = 128
DIN = DM + DB
DFF = 2048
T = 8192
BLK = 256
CAP = T + NE * BLK
NBLK = CAP // BLK
NBLK_PAD = 48

L = 16
NC = 2
NS = 16
NW = NC * NS
TPW = T // NW
CH = 64
NCH = TPW // CH
CH2 = 32
NCH2 = TPW // CH2

_sc_mesh = plsc.VectorSubcoreMesh(core_axis_name="c", subcore_axis_name="s")


@functools.partial(
    pl.kernel,
    out_type=(
        jax.ShapeDtypeStruct((CAP, DM), jnp.float32),
        jax.ShapeDtypeStruct((CAP, DBP), jnp.float32),
        jax.ShapeDtypeStruct((T,), jnp.int32),
        jax.ShapeDtypeStruct((NBLK_PAD,), jnp.int32),
        jax.ShapeDtypeStruct((L,), jnp.int32),
    ),
    mesh=_sc_mesh,
    compiler_params=pltpu.CompilerParams(needs_layout_passes=False),
    scratch_types=[
        pltpu.VMEM((2 * TPW,), jnp.int32),
        pltpu.VMEM((TPW,), jnp.int32),
        pltpu.VMEM((NCH2, CH2), jnp.int32),
        pltpu.VMEM((L,), jnp.int32),
        pltpu.VMEM((NW * L,), jnp.int32),
        pltpu.VMEM((L,), jnp.int32),
        pltpu.VMEM((L,), jnp.int32),
        pltpu.VMEM((NBLK_PAD,), jnp.int32),
        pltpu.VMEM((TPW, DBP), jnp.float32),
        pltpu.VMEM((CH2, DM), jnp.float32),
        pltpu.VMEM((CH2, DM), jnp.float32),
        pltpu.VMEM_SHARED((NW * L,), jnp.int32),
        pltpu.SemaphoreType.DMA,
        pltpu.SemaphoreType.DMA,
        pltpu.SemaphoreType.DMA,
        pltpu.SemaphoreType.DMA,
        pltpu.SemaphoreType.DMA,
        pltpu.SemaphoreType.DMA,
        pltpu.SemaphoreType.DMA,
    ],
)
def _sc_dispatch(hid_hbm, pos_hbm, beh_hbm, bt_hbm,
                 xh_hbm, xb_hbm, dest_hbm, bexp_hbm, meta_hbm,
                 pos_v, behi_v, dest_v, cnt_v, allcnt_v, base_v, ends_v,
                 bexp_v, behrows_v, hidrows0, hidrows1, counts_sh,
                 sem, rsem0, rsem1, wsem0, wsem1, dsem, bsem):
    c = lax.axis_index("c")
    s = lax.axis_index("s")
    g = s * 2 + c
    lanes = lax.broadcasted_iota(jnp.int32, (L,), 0)
    zeros = jnp.zeros((L,), jnp.int32)

    pltpu.sync_copy(pos_hbm.at[pl.ds(s * 2 * TPW, 2 * TPW)], pos_v)
    for half in range(2):
        cnt = zeros
        for i in range(TPW // L):
            p = pos_v[pl.ds(half * TPW + i * L, L)]
            for e in range(NE):
                cpop = jnp.sum((p == e).astype(jnp.int32))
                cnt = cnt + jnp.where(
                    lanes == e, lax.broadcast_in_dim(cpop, (L,), ()), 0)
        cnt_v[...] = cnt
        pltpu.sync_copy(cnt_v, counts_sh.at[pl.ds((2 * s + half) * L, L)])
    plsc.subcore_barrier()
    pltpu.sync_copy(counts_sh, allcnt_v)

    gv = lax.broadcast_in_dim(g, (L,), ())
    total = zeros
    prior = zeros
    for r in range(NW):
        row = allcnt_v[pl.ds(r * L, L)]
        total = total + row
        prior = prior + jnp.where(jnp.full((L,), r, jnp.int32) < gv, row, zeros)

    pcnt = ((total + (BLK - 1)) >> 8) << 8
    ends = plsc.cumsum(pcnt)
    starts = ends - pcnt
    base_v[...] = starts + prior
    ends_v[...] = ends

    for i in range(TPW // L):
        p = pos_v[pl.ds(c * TPW + i * L, L)]
        bofp = plsc.load_gather(base_v, [p])
        rank = zeros
        for e in range(NE):
            m = p == e
            cs = plsc.cumsum(m.astype(jnp.int32))
            rank = jnp.where(m, cs - 1, rank)
            cpop = lax.broadcast_in_dim(jnp.sum(m.astype(jnp.int32)), (L,), ())
            plsc.addupdate_scatter(
                base_v, [jnp.full((L,), e, jnp.int32)], cpop, mask=lanes == 0)
        dest_v[i // (CH2 // L), pl.ds((i % (CH2 // L)) * L, L)] = bofp + rank

    tok0 = g * TPW
    pltpu.sync_copy(beh_hbm.at[pl.ds(tok0, TPW)], behi_v)
    bgets = [
        pltpu.async_copy(
            bt_hbm.at[behi_v.at[pl.ds(h * 128, 128)]],
            behrows_v.at[pl.ds(h * 128, 128)], sem)
        for h in range(TPW // 128)
    ]
    dwrites = [
        pltpu.async_copy(
            dest_v.at[j], dest_hbm.at[pl.ds(tok0 + j * CH2, CH2)], dsem)
        for j in range(NCH2)
    ]
    for bget in bgets:
        bget.wait()
    bscat = [
        pltpu.async_copy(
            behrows_v.at[pl.ds(j * CH2, CH2)], xb_hbm.at[dest_v.at[j]], bsem)
        for j in range(NCH2)
    ]

    bufs = (hidrows0, hidrows1)
    rsems = (rsem0, rsem1)
    wsems = (wsem0, wsem1)
    reads = [None] * NCH2
    writes = [None] * NCH2
    reads[0] = pltpu.async_copy(
        hid_hbm.at[pl.ds(tok0, CH2)], bufs[0], rsems[0])
    for j in range(NCH2):
        b = j % 2
        reads[j].wait()
        if j + 1 < NCH2:
            if j >= 1:
                writes[j - 1].wait()
            reads[j + 1] = pltpu.async_copy(
                hid_hbm.at[pl.ds(tok0 + (j + 1) * CH2, CH2)],
                bufs[(j + 1) % 2], rsems[(j + 1) % 2])
        writes[j] = pltpu.async_copy(
            bufs[b], xh_hbm.at[dest_v.at[j]], wsems[b])
    writes[NCH2 - 2].wait()
    writes[NCH2 - 1].wait()
    for w in bscat:
        w.wait()
    for w in dwrites:
        w.wait()

    for k in range(NBLK_PAD // L):
        blk = (lanes + k * L) * BLK
        cntk = zeros
        for e in range(NE):
            ende = jnp.take_along_axis(
                ends, jnp.full((L,), e, jnp.int32), axis=0,
                mode="promise_in_bounds")
            cntk = cntk + jnp.where(ende <= blk, 1, 0)
        bexp_v[pl.ds(k * L, L)] = jnp.minimum(cntk, NE - 1)

    @pl.when((c == 0) & (s == 0))
    def _():
        pltpu.sync_copy(bexp_v, bexp_hbm)
        pltpu.sync_copy(ends_v, meta_hbm)


@functools.partial(
    pl.kernel,
    out_type=jax.ShapeDtypeStruct((T, DM), jnp.float32),
    mesh=_sc_mesh,
    compiler_params=pltpu.CompilerParams(needs_layout_passes=False),
    scratch_types=[
        pltpu.VMEM((TPW,), jnp.int32),
        pltpu.VMEM((CH2, DM), jnp.float32),
        pltpu.VMEM((CH2, DM), jnp.float32),
        pltpu.SemaphoreType.DMA,
        pltpu.SemaphoreType.DMA,
        pltpu.SemaphoreType.DMA,
        pltpu.SemaphoreType.DMA,
    ],
)
def _sc_combine(osort_hbm, dest_hbm, out_hbm, dest_v, rows0, rows1,
                rsem0, rsem1, wsem0, wsem1):
    c = lax.axis_index("c")
    s = lax.axis_index("s")
    tok0 = (s * 2 + c) * TPW
    pltpu.sync_copy(dest_hbm.at[pl.ds(tok0, TPW)], dest_v)
    bufs = (rows0, rows1)
    rsems = (rsem0, rsem1)
    wsems = (wsem0, wsem1)
    reads = [None] * NCH2
    writes = [None] * NCH2
    reads[0] = pltpu.async_copy(
        osort_hbm.at[dest_v.at[pl.ds(0, CH2)]], bufs[0], rsems[0])
    for j in range(NCH2):
        b = j % 2
        reads[j].wait()
        if j + 1 < NCH2:
            if j >= 1:
                writes[j - 1].wait()
            reads[j + 1] = pltpu.async_copy(
                osort_hbm.at[dest_v.at[pl.ds((j + 1) * CH2, CH2)]],
                bufs[(j + 1) % 2], rsems[(j + 1) % 2])
        writes[j] = pltpu.async_copy(
            bufs[b], out_hbm.at[pl.ds(tok0 + j * CH2, CH2)], wsems[b])
    writes[NCH2 - 2].wait()
    writes[NCH2 - 1].wait()


def _mlp_body(be_ref, meta_ref, xh_ref, xb_ref, wg_ref, wu_ref, wd_ref,
              o_ref):
    del be_ref
    nused = meta_ref[NE - 1] >> 8

    @pl.when(pl.program_id(0) < nused)
    def _():
        _mlp_compute(xh_ref, xb_ref, wg_ref, wu_ref, wd_ref, o_ref)


def _mlp_compute(xh_ref, xb_ref, wg_ref, wu_ref, wd_ref, o_ref):
    xh = xh_ref[...].astype(jnp.bfloat16)
    xb = xb_ref[:, :DB].astype(jnp.bfloat16)
    wg = wg_ref[0].astype(jnp.bfloat16)
    wu = wu_ref[0].astype(jnp.bfloat16)
    wd = wd_ref[0].astype(jnp.bfloat16)
    g = jnp.dot(xh, wg[:DM], preferred_element_type=jnp.float32)
    g = g + jnp.dot(xb, wg[DM:], preferred_element_type=jnp.float32)
    u = jnp.dot(xh, wu[:DM], preferred_element_type=jnp.float32)
    u = u + jnp.dot(xb, wu[DM:], preferred_element_type=jnp.float32)
    a = (g * jax.nn.sigmoid(g) * u).astype(jnp.bfloat16)
    o_ref[...] = jnp.dot(a, wd, preferred_element_type=jnp.float32)


def _grouped_mlp(block_expert, meta, x_hid, x_beh, wg, wu, wd):
    grid_spec = pltpu.PrefetchScalarGridSpec(
        num_scalar_prefetch=2,
        grid=(NBLK,),
        in_specs=[
            pl.BlockSpec((BLK, DM), lambda b, be, me: (b, 0)),
            pl.BlockSpec((BLK, DBP), lambda b, be, me: (b, 0)),
            pl.BlockSpec((1, DIN, DFF), lambda b, be, me: (be[b], 0, 0)),
            pl.BlockSpec((1, DIN, DFF), lambda b, be, me: (be[b], 0, 0)),
            pl.BlockSpec((1, DFF, DM), lambda b, be, me: (be[b], 0, 0)),
        ],
        out_specs=pl.BlockSpec((BLK, DM), lambda b, be, me: (b, 0)),
    )
    return pl.pallas_call(
        _mlp_body,
        grid_spec=grid_spec,
        out_shape=jax.ShapeDtypeStruct((CAP, DM), jnp.float32),
        compiler_params=pltpu.CompilerParams(
            vmem_limit_bytes=110 * 1024 * 1024),
    )(block_expert, meta, x_hid, x_beh, wg, wu, wd)


def kernel(hidden_states, position_index, behavior_index, behavior_table,
           W_gate, W_up, W_down):
    pos = position_index.astype(jnp.int32)
    beh = behavior_index.astype(jnp.int32)
    bt_pad = jnp.pad(behavior_table, ((0, 0), (0, DBP - DB)))
    x_hid, x_beh, dest, block_expert, meta = _sc_dispatch(
        hidden_states, pos, beh, bt_pad)
    out_sorted = _grouped_mlp(
        block_expert, meta, x_hid, x_beh, W_gate, W_up, W_down)
    return _sc_combine(out_sorted, dest)


# speedup vs baseline: 1.0600x; 1.0002x over previous
import functools

import jax
import jax.numpy as jnp
from jax import lax
from jax.experimental import pallas as pl
from jax.experimental.pallas import tpu as pltpu
from jax.experimental.pallas import tpu_sc as plsc

NE = 8
DM = 1024
DB = 64
DBP = 128
DIN = DM + DB
DFF = 2048
T = 8192
BLK = 256
CAP = T + NE * BLK
NBLK = CAP // BLK
NBLK_PAD = 48

L = 16
NC = 2
NS = 16
NW = NC * NS
TPW = T // NW
CH2 = 32
NCH2 = TPW // CH2

_sc_mesh = plsc.VectorSubcoreMesh(core_axis_name="c", subcore_axis_name="s")


@functools.partial(
    pl.kernel,
    out_type=(
        jax.ShapeDtypeStruct((CAP, DM), jnp.float32),
        jax.ShapeDtypeStruct((CAP, DBP), jnp.float32),
        jax.ShapeDtypeStruct((T,), jnp.int32),
        jax.ShapeDtypeStruct((NBLK_PAD,), jnp.int32),
        jax.ShapeDtypeStruct((L,), jnp.int32),
    ),
    mesh=_sc_mesh,
    compiler_params=pltpu.CompilerParams(needs_layout_passes=False),
    scratch_types=[
        pltpu.VMEM((2 * TPW,), jnp.int32),
        pltpu.VMEM((TPW,), jnp.int32),
        pltpu.VMEM((NCH2, CH2), jnp.int32),
        pltpu.VMEM((L,), jnp.int32),
        pltpu.VMEM((NW * L,), jnp.int32),
        pltpu.VMEM((L,), jnp.int32),
        pltpu.VMEM((L,), jnp.int32),
        pltpu.VMEM((NBLK_PAD,), jnp.int32),
        pltpu.VMEM((TPW, DBP), jnp.float32),
        pltpu.VMEM((CH2, DM), jnp.float32),
        pltpu.VMEM((CH2, DM), jnp.float32),
        pltpu.VMEM_SHARED((NW * L,), jnp.int32),
        pltpu.SemaphoreType.DMA,
        pltpu.SemaphoreType.DMA,
        pltpu.SemaphoreType.DMA,
        pltpu.SemaphoreType.DMA,
        pltpu.SemaphoreType.DMA,
        pltpu.SemaphoreType.DMA,
        pltpu.SemaphoreType.DMA,
    ],
)
def _sc_dispatch(hid_hbm, pos_hbm, beh_hbm, bt_hbm,
                 xh_hbm, xb_hbm, dest_hbm, bexp_hbm, meta_hbm,
                 pos_v, behi_v, dest_v, cnt_v, allcnt_v, base_v, ends_v,
                 bexp_v, behrows_v, hidrows0, hidrows1, counts_sh,
                 sem, rsem0, rsem1, wsem0, wsem1, dsem, bsem):
    c = lax.axis_index("c")
    s = lax.axis_index("s")
    g = s * 2 + c
    lanes = lax.broadcasted_iota(jnp.int32, (L,), 0)
    zeros = jnp.zeros((L,), jnp.int32)

    pltpu.sync_copy(pos_hbm.at[pl.ds(s * 2 * TPW, 2 * TPW)], pos_v)
    for half in range(2):
        cnt = zeros
        for i in range(TPW // L):
            p = pos_v[pl.ds(half * TPW + i * L, L)]
            for e in range(NE):
                cpop = jnp.sum((p == e).astype(jnp.int32))
                cnt = cnt + jnp.where(
                    lanes == e, lax.broadcast_in_dim(cpop, (L,), ()), 0)
        cnt_v[...] = cnt
        pltpu.sync_copy(cnt_v, counts_sh.at[pl.ds((2 * s + half) * L, L)])
    plsc.subcore_barrier()
    pltpu.sync_copy(counts_sh, allcnt_v)

    gv = lax.broadcast_in_dim(g, (L,), ())
    total = zeros
    prior = zeros
    for r in range(NW):
        row = allcnt_v[pl.ds(r * L, L)]
        total = total + row
        prior = prior + jnp.where(jnp.full((L,), r, jnp.int32) < gv, row, zeros)

    pcnt = ((total + (BLK - 1)) >> 8) << 8
    ends = plsc.cumsum(pcnt)
    starts = ends - pcnt
    base_v[...] = starts + prior
    ends_v[...] = ends

    for i in range(TPW // L):
        p = pos_v[pl.ds(c * TPW + i * L, L)]
        bofp = plsc.load_gather(base_v, [p])
        rank = zeros
        for e in range(NE):
            m = p == e
            cs = plsc.cumsum(m.astype(jnp.int32))
            rank = jnp.where(m, cs - 1, rank)
            cpop = lax.broadcast_in_dim(jnp.sum(m.astype(jnp.int32)), (L,), ())
            plsc.addupdate_scatter(
                base_v, [jnp.full((L,), e, jnp.int32)], cpop, mask=lanes == 0)
        dest_v[i // (CH2 // L), pl.ds((i % (CH2 // L)) * L, L)] = bofp + rank

    tok0 = g * TPW
    pltpu.sync_copy(beh_hbm.at[pl.ds(tok0, TPW)], behi_v)
    bgets = [
        pltpu.async_copy(
            bt_hbm.at[behi_v.at[pl.ds(h * 128, 128)]],
            behrows_v.at[pl.ds(h * 128, 128)], sem)
        for h in range(TPW // 128)
    ]
    dwrites = [
        pltpu.async_copy(
            dest_v.at[j], dest_hbm.at[pl.ds(tok0 + j * CH2, CH2)], dsem)
        for j in range(NCH2)
    ]
    for bget in bgets:
        bget.wait()
    bscat = [
        pltpu.async_copy(
            behrows_v.at[pl.ds(j * CH2, CH2)], xb_hbm.at[dest_v.at[j]], bsem)
        for j in range(NCH2)
    ]

    bufs = (hidrows0, hidrows1)
    rsems = (rsem0, rsem1)
    wsems = (wsem0, wsem1)
    reads = [None] * NCH2
    writes = [None] * NCH2
    reads[0] = pltpu.async_copy(
        hid_hbm.at[pl.ds(tok0, CH2)], bufs[0], rsems[0])
    for j in range(NCH2):
        b = j % 2
        reads[j].wait()
        if j + 1 < NCH2:
            if j >= 1:
                writes[j - 1].wait()
            reads[j + 1] = pltpu.async_copy(
                hid_hbm.at[pl.ds(tok0 + (j + 1) * CH2, CH2)],
                bufs[(j + 1) % 2], rsems[(j + 1) % 2])
        writes[j] = pltpu.async_copy(
            bufs[b], xh_hbm.at[dest_v.at[j]], wsems[b])
    writes[NCH2 - 2].wait()
    writes[NCH2 - 1].wait()
    for w in bscat:
        w.wait()
    for w in dwrites:
        w.wait()

    for k in range(NBLK_PAD // L):
        blk = (lanes + k * L) * BLK
        cntk = zeros
        for e in range(NE):
            ende = jnp.take_along_axis(
                ends, jnp.full((L,), e, jnp.int32), axis=0,
                mode="promise_in_bounds")
            cntk = cntk + jnp.where(ende <= blk, 1, 0)
        bexp_v[pl.ds(k * L, L)] = jnp.minimum(cntk, NE - 1)

    @pl.when((c == 0) & (s == 0))
    def _():
        pltpu.sync_copy(bexp_v, bexp_hbm)
        pltpu.sync_copy(ends_v, meta_hbm)


@functools.partial(
    pl.kernel,
    out_type=jax.ShapeDtypeStruct((T, DM), jnp.float32),
    mesh=_sc_mesh,
    compiler_params=pltpu.CompilerParams(needs_layout_passes=False),
    scratch_types=[
        pltpu.VMEM((TPW,), jnp.int32),
        pltpu.VMEM((CH2, DM), jnp.float32),
        pltpu.VMEM((CH2, DM), jnp.float32),
        pltpu.SemaphoreType.DMA,
        pltpu.SemaphoreType.DMA,
        pltpu.SemaphoreType.DMA,
        pltpu.SemaphoreType.DMA,
    ],
)
def _sc_combine(osort_hbm, dest_hbm, out_hbm, dest_v, rows0, rows1,
                rsem0, rsem1, wsem0, wsem1):
    c = lax.axis_index("c")
    s = lax.axis_index("s")
    tok0 = (s * 2 + c) * TPW
    pltpu.sync_copy(dest_hbm.at[pl.ds(tok0, TPW)], dest_v)
    bufs = (rows0, rows1)
    rsems = (rsem0, rsem1)
    wsems = (wsem0, wsem1)
    reads = [None] * NCH2
    writes = [None] * NCH2
    reads[0] = pltpu.async_copy(
        osort_hbm.at[dest_v.at[pl.ds(0, CH2)]], bufs[0], rsems[0])
    for j in range(NCH2):
        b = j % 2
        reads[j].wait()
        if j + 1 < NCH2:
            if j >= 1:
                writes[j - 1].wait()
            reads[j + 1] = pltpu.async_copy(
                osort_hbm.at[dest_v.at[pl.ds((j + 1) * CH2, CH2)]],
                bufs[(j + 1) % 2], rsems[(j + 1) % 2])
        writes[j] = pltpu.async_copy(
            bufs[b], out_hbm.at[pl.ds(tok0 + j * CH2, CH2)], wsems[b])
    writes[NCH2 - 2].wait()
    writes[NCH2 - 1].wait()


def _mlp_body(be_ref, meta_ref, xh_ref, xb_ref, wg_ref, wu_ref, wd_ref,
              o_ref):
    del be_ref
    nused = meta_ref[NE - 1] >> 8

    @pl.when(pl.program_id(0) < nused)
    def _():
        _mlp_compute(xh_ref, xb_ref, wg_ref, wu_ref, wd_ref, o_ref)


def _mlp_compute(xh_ref, xb_ref, wg_ref, wu_ref, wd_ref, o_ref):
    xh = xh_ref[...].astype(jnp.bfloat16)
    xb = xb_ref[:, :DB].astype(jnp.bfloat16)
    wg = wg_ref[0].astype(jnp.bfloat16)
    wu = wu_ref[0].astype(jnp.bfloat16)
    wd = wd_ref[0].astype(jnp.bfloat16)
    g = jnp.dot(xh, wg[:DM], preferred_element_type=jnp.float32)
    g = g + jnp.dot(xb, wg[DM:], preferred_element_type=jnp.float32)
    u = jnp.dot(xh, wu[:DM], preferred_element_type=jnp.float32)
    u = u + jnp.dot(xb, wu[DM:], preferred_element_type=jnp.float32)
    a = (g * jax.nn.sigmoid(g) * u).astype(jnp.bfloat16)
    o_ref[...] = jnp.dot(a, wd, preferred_element_type=jnp.float32)


def _grouped_mlp(block_expert, meta, x_hid, x_beh, wg, wu, wd):
    grid_spec = pltpu.PrefetchScalarGridSpec(
        num_scalar_prefetch=2,
        grid=(NBLK,),
        in_specs=[
            pl.BlockSpec((BLK, DM), lambda b, be, me: (b, 0)),
            pl.BlockSpec((BLK, DBP), lambda b, be, me: (b, 0)),
            pl.BlockSpec((1, DIN, DFF), lambda b, be, me: (be[b], 0, 0)),
            pl.BlockSpec((1, DIN, DFF), lambda b, be, me: (be[b], 0, 0)),
            pl.BlockSpec((1, DFF, DM), lambda b, be, me: (be[b], 0, 0)),
        ],
        out_specs=pl.BlockSpec((BLK, DM), lambda b, be, me: (b, 0)),
    )
    return pl.pallas_call(
        _mlp_body,
        grid_spec=grid_spec,
        out_shape=jax.ShapeDtypeStruct((CAP, DM), jnp.float32),
        compiler_params=pltpu.CompilerParams(
            vmem_limit_bytes=110 * 1024 * 1024),
    )(block_expert, meta, x_hid, x_beh, wg, wu, wd)


def kernel(hidden_states, position_index, behavior_index, behavior_table,
           W_gate, W_up, W_down):
    pos = position_index.astype(jnp.int32)
    beh = behavior_index.astype(jnp.int32)
    bt_pad = jnp.pad(behavior_table, ((0, 0), (0, DBP - DB)))
    x_hid, x_beh, dest, block_expert, meta = _sc_dispatch(
        hidden_states, pos, beh, bt_pad)
    out_sorted = _grouped_mlp(
        block_expert, meta, x_hid, x_beh, W_gate, W_up, W_down)
    return _sc_combine(out_sorted, dest)
